# baseline (speedup 1.0000x reference)
import functools

import jax
import jax.numpy as jnp
from jax.experimental import pallas as pl


def _rvq_body(num_q, x_ref, cb_ref, out_ref, idx_ref):
    resid = x_ref[...]
    acc = jnp.zeros_like(resid)
    t, d = resid.shape
    k = cb_ref.shape[1]
    iota_k = jax.lax.broadcasted_iota(jnp.int32, (t, k), 1)
    for q in range(num_q):
        cb = cb_ref[q]
        r2 = jnp.sum(resid * resid, axis=-1, keepdims=True)
        c2 = jnp.sum(cb * cb, axis=-1)
        dots = jax.lax.dot_general(
            resid, cb, (((1,), (1,)), ((), ())),
            preferred_element_type=jnp.float32)
        dist = r2 - 2.0 * dots + c2[None, :]
        idx = jnp.argmin(dist, axis=-1).astype(jnp.int32)
        hi = cb.astype(jnp.bfloat16)
        rem = cb - hi.astype(jnp.float32)
        mid = rem.astype(jnp.bfloat16)
        lo = (rem - mid.astype(jnp.float32)).astype(jnp.bfloat16)
        onehot = (iota_k == idx[:, None]).astype(jnp.bfloat16)
        dn = (((1,), (0,)), ((), ()))
        quant = ((jax.lax.dot_general(
                      onehot, hi, dn, preferred_element_type=jnp.float32)
                  + jax.lax.dot_general(
                      onehot, mid, dn, preferred_element_type=jnp.float32))
                 + jax.lax.dot_general(
                      onehot, lo, dn, preferred_element_type=jnp.float32))
        s = resid + (quant - resid)
        acc = acc + s
        resid = resid - s
        idx_ref[q, :] = idx
    out_ref[...] = acc


def kernel(x, codebooks):
    b, n, d = x.shape
    num_q, k, _ = codebooks.shape
    tokens = b * n
    t = 256
    xf = x.reshape(tokens, d)
    grid = (tokens // t,)
    out, idx = pl.pallas_call(
        functools.partial(_rvq_body, num_q),
        grid=grid,
        in_specs=[
            pl.BlockSpec((t, d), lambda i: (i, 0)),
            pl.BlockSpec((num_q, k, d), lambda i: (0, 0, 0)),
        ],
        out_specs=[
            pl.BlockSpec((t, d), lambda i: (i, 0)),
            pl.BlockSpec((num_q, t), lambda i: (0, i)),
        ],
        out_shape=[
            jax.ShapeDtypeStruct((tokens, d), jnp.float32),
            jax.ShapeDtypeStruct((num_q, tokens), jnp.int32),
        ],
    )(xf, codebooks)
    return out.reshape(b, n, d), idx.T.reshape(b, n, num_q)


# speedup vs baseline: 1.9703x; 1.9703x over previous
import functools

import jax
import jax.numpy as jnp
from jax import lax
from jax.experimental import pallas as pl
from jax.experimental.pallas import tpu as pltpu, tpu_sc as plsc

NC, NS = 2, 16
NW = NC * NS
GCH = 128


def _sc_gather(table, idx):
    b = idx.shape[0]
    dd = table.shape[1]
    b_per_w = b // NW
    nch = b_per_w // GCH
    mesh = plsc.VectorSubcoreMesh(core_axis_name="c", subcore_axis_name="s")

    @functools.partial(
        pl.kernel, mesh=mesh,
        out_type=jax.ShapeDtypeStruct((b, dd), jnp.float32),
        scratch_types=[
            pltpu.VMEM((GCH,), jnp.int32),
            pltpu.VMEM((b_per_w, dd), jnp.float32),
            pltpu.SemaphoreType.DMA,
        ],
    )
    def k(table_hbm, idx_hbm, out_hbm, idx_v, rows_v, sem):
        wid = lax.axis_index("s") * NC + lax.axis_index("c")
        base = wid * b_per_w
        for c in range(nch):
            pltpu.sync_copy(idx_hbm.at[pl.ds(base + c * GCH, GCH)], idx_v)
            pltpu.async_copy(
                table_hbm.at[idx_v],
                rows_v.at[pl.ds(c * GCH, GCH)], sem).wait()
        pltpu.sync_copy(rows_v, out_hbm.at[pl.ds(base, b_per_w)])

    return k(table, idx)


def _tc_body(last, x_ref, q_ref, a_ref, cb_ref, r_ref, acc_ref, idx_ref):
    resid_in = x_ref[...]
    s = resid_in + (q_ref[...] - resid_in)
    acc = a_ref[...] + s
    resid = resid_in - s
    r_ref[...] = resid
    acc_ref[...] = acc
    if not last:
        cb = cb_ref[0]
        r2 = jnp.sum(resid * resid, axis=-1, keepdims=True)
        c2 = jnp.sum(cb * cb, axis=-1)
        dots = jax.lax.dot_general(
            resid, cb, (((1,), (1,)), ((), ())),
            preferred_element_type=jnp.float32)
        dist = r2 - 2.0 * dots + c2[None, :]
        idx_ref[0, :] = jnp.argmin(dist, axis=-1).astype(jnp.int32)
    else:
        idx_ref[0, :] = jnp.zeros_like(idx_ref[0, :])


def _tc_layer(resid, quant, acc, cb_q, t, last=False):
    tokens, d = resid.shape
    k = cb_q.shape[1]
    r, a, idx = pl.pallas_call(
        functools.partial(_tc_body, last),
        grid=(tokens // t,),
        in_specs=[
            pl.BlockSpec((t, d), lambda i: (i, 0)),
            pl.BlockSpec((t, d), lambda i: (i, 0)),
            pl.BlockSpec((t, d), lambda i: (i, 0)),
            pl.BlockSpec((1, k, d), lambda i: (0, 0, 0)),
        ],
        out_specs=[
            pl.BlockSpec((t, d), lambda i: (i, 0)),
            pl.BlockSpec((t, d), lambda i: (i, 0)),
            pl.BlockSpec((1, t), lambda i: (0, i)),
        ],
        out_shape=[
            jax.ShapeDtypeStruct((tokens, d), jnp.float32),
            jax.ShapeDtypeStruct((tokens, d), jnp.float32),
            jax.ShapeDtypeStruct((1, tokens), jnp.int32),
        ],
    )(resid, quant, acc, cb_q)
    return r, a, idx[0]


def kernel(x, codebooks):
    b, n, d = x.shape
    num_q, k, _ = codebooks.shape
    tokens = b * n
    t = 256
    xf = x.reshape(tokens, d)
    cb_pad = jnp.pad(codebooks, ((0, 0), (0, 0), (0, 128 - d)))
    resid = xf
    quant = jnp.zeros_like(xf)
    acc = jnp.zeros_like(xf)
    idxs = []
    for q in range(num_q):
        resid, acc, idx = _tc_layer(
            resid, quant, acc, codebooks[q:q + 1], t)
        idxs.append(idx)
        quant = _sc_gather(cb_pad[q], idx)[:, :d]
    resid, acc, _ = _tc_layer(
        resid, quant, acc, codebooks[:1], t, last=True)
    all_idx = jnp.stack(idxs, axis=-1)
    return acc.reshape(b, n, d), all_idx.reshape(b, n, num_q)


# speedup vs baseline: 2.4078x; 1.2221x over previous
import functools

import jax
import jax.numpy as jnp
from jax import lax
from jax.experimental import pallas as pl
from jax.experimental.pallas import tpu as pltpu, tpu_sc as plsc

NC, NS = 2, 16
NW = NC * NS
GCH = 128


def _sc_gather(table, idx):
    b = idx.shape[0]
    dd = table.shape[1]
    b_per_w = b // NW
    nch = b_per_w // GCH
    mesh = plsc.VectorSubcoreMesh(core_axis_name="c", subcore_axis_name="s")

    @functools.partial(
        pl.kernel, mesh=mesh,
        out_type=jax.ShapeDtypeStruct((b, dd), jnp.float32),
        scratch_types=[
            pltpu.VMEM((GCH,), jnp.int32),
            pltpu.VMEM((b_per_w, dd), jnp.float32),
            pltpu.SemaphoreType.DMA,
        ],
    )
    def k(table_hbm, idx_hbm, out_hbm, idx_v, rows_v, sem):
        wid = lax.axis_index("s") * NC + lax.axis_index("c")
        base = wid * b_per_w
        for c in range(nch):
            pltpu.sync_copy(idx_hbm.at[pl.ds(base + c * GCH, GCH)], idx_v)
            pltpu.async_copy(
                table_hbm.at[idx_v],
                rows_v.at[pl.ds(c * GCH, GCH)], sem).wait()
        pltpu.sync_copy(rows_v, out_hbm.at[pl.ds(base, b_per_w)])

    return k(table, idx)


def _tc_body(last, x_ref, q_ref, a_ref, cb_ref, r_ref, acc_ref, idx_ref):
    resid_in = x_ref[...]
    s = resid_in + (q_ref[...] - resid_in)
    acc = a_ref[...] + s
    resid = resid_in - s
    r_ref[...] = resid
    acc_ref[...] = acc
    if not last:
        cb = cb_ref[0]
        r2 = jnp.sum(resid * resid, axis=-1, keepdims=True)
        c2 = jnp.sum(cb * cb, axis=-1)
        dots = jax.lax.dot_general(
            resid, cb, (((1,), (1,)), ((), ())),
            preferred_element_type=jnp.float32)
        dist = r2 - 2.0 * dots + c2[None, :]
        idx_ref[0, :] = jnp.argmin(dist, axis=-1).astype(jnp.int32)
    else:
        idx_ref[0, :] = jnp.zeros_like(idx_ref[0, :])


def _tc_layer(resid, quant, acc, cb_q, t, last=False):
    tokens, d = resid.shape
    k = cb_q.shape[1]
    r, a, idx = pl.pallas_call(
        functools.partial(_tc_body, last),
        grid=(tokens // t,),
        in_specs=[
            pl.BlockSpec((t, d), lambda i: (i, 0)),
            pl.BlockSpec((t, d), lambda i: (i, 0)),
            pl.BlockSpec((t, d), lambda i: (i, 0)),
            pl.BlockSpec((1, k, d), lambda i: (0, 0, 0)),
        ],
        out_specs=[
            pl.BlockSpec((t, d), lambda i: (i, 0)),
            pl.BlockSpec((t, d), lambda i: (i, 0)),
            pl.BlockSpec((1, t), lambda i: (0, i)),
        ],
        out_shape=[
            jax.ShapeDtypeStruct((tokens, d), jnp.float32),
            jax.ShapeDtypeStruct((tokens, d), jnp.float32),
            jax.ShapeDtypeStruct((1, tokens), jnp.int32),
        ],
    )(resid, quant, acc, cb_q)
    return r, a, idx[0]


def kernel(x, codebooks):
    b, n, d = x.shape
    num_q, k, _ = codebooks.shape
    tokens = b * n
    t = 512
    xf = x.reshape(tokens, d)
    cb_pad = jnp.pad(codebooks, ((0, 0), (0, 0), (0, 128 - d)))
    resid = xf
    quant = jnp.zeros_like(xf)
    acc = jnp.zeros_like(xf)
    idxs = []
    for q in range(num_q):
        resid, acc, idx = _tc_layer(
            resid, quant, acc, codebooks[q:q + 1], t)
        idxs.append(idx)
        quant = _sc_gather(cb_pad[q], idx)[:, :d]
    resid, acc, _ = _tc_layer(
        resid, quant, acc, codebooks[:1], t, last=True)
    all_idx = jnp.stack(idxs, axis=-1)
    return acc.reshape(b, n, d), all_idx.reshape(b, n, num_q)
